# Initial kernel scaffold; baseline (speedup 1.0000x reference)
import functools

import jax
import jax.numpy as jnp
from jax import lax
from jax.experimental import pallas as pl
from jax.experimental.pallas import tpu as pltpu
from jax.experimental.pallas import tpu_sc as plsc

EMBED_DIM = 64
NUM_CORES = 2
NUM_SUBCORES = 16
NUM_WORKERS = NUM_CORES * NUM_SUBCORES
GATHER_ROWS = 128
NBUF = 4


@functools.partial(jax.jit, static_argnames=("b_per_w", "steps"))
def _sc_gather(idx3, table, *, b_per_w, steps):
    total = NUM_WORKERS * b_per_w
    mesh = plsc.VectorSubcoreMesh(core_axis_name="c", subcore_axis_name="s")

    @functools.partial(
        pl.kernel,
        mesh=mesh,
        out_type=jax.ShapeDtypeStruct((total, EMBED_DIM), jnp.float32),
        scratch_types=[
            pltpu.VMEM((steps, GATHER_ROWS), jnp.int32),
            pltpu.VMEM((NBUF, GATHER_ROWS, EMBED_DIM), jnp.float32),
        ] + [pltpu.SemaphoreType.DMA] * NBUF,
    )
    def k(idx_hbm, table_hbm, out_hbm, idx_v, rows_v, *sems):
        wid = lax.axis_index("s") * NUM_CORES + lax.axis_index("c")
        base = wid * b_per_w
        pltpu.sync_copy(idx_hbm.at[wid], idx_v)

        def start(g, b):
            pltpu.async_copy(table_hbm.at[idx_v.at[g]], rows_v.at[b], sems[b])

        def wait(b):
            pltpu.make_async_copy(
                table_hbm.at[pl.ds(0, GATHER_ROWS)], rows_v.at[b], sems[b]
            ).wait()

        def drain(g, b):
            wait(b)
            pltpu.sync_copy(
                rows_v.at[b],
                out_hbm.at[pl.ds(base + g * GATHER_ROWS, GATHER_ROWS)],
            )

        for b in range(NBUF):
            start(b, b)

        def outer(i, carry):
            g0 = i * NBUF
            for b in range(NBUF):
                drain(g0 + b, b)
                start(g0 + b + NBUF, b)
            return carry

        lax.fori_loop(0, steps // NBUF - 1, outer, 0)
        for b in range(NBUF):
            drain(steps - NBUF + b, b)

    return k(idx3, table)


def kernel(x, table):
    batch, hist = x.shape
    total = batch * hist
    b_per_w = total // NUM_WORKERS
    steps = b_per_w // GATHER_ROWS
    idx3 = x.reshape(NUM_WORKERS, steps, GATHER_ROWS).astype(jnp.int32)
    out = _sc_gather(idx3, table, b_per_w=b_per_w, steps=steps)
    return out.reshape(batch, hist, EMBED_DIM)


# speedup vs baseline: 1.8793x; 1.8793x over previous
import functools

import jax
import jax.numpy as jnp
from jax import lax
from jax.experimental import pallas as pl
from jax.experimental.pallas import tpu as pltpu
from jax.experimental.pallas import tpu_sc as plsc

EMBED_DIM = 64
NUM_CORES = 2
NUM_SUBCORES = 16
NUM_WORKERS = NUM_CORES * NUM_SUBCORES
GATHER_ROWS = 128
NBUF = 4


@functools.partial(jax.jit, static_argnames=("b_per_w", "steps"))
def _sc_gather(idx3, table, *, b_per_w, steps):
    total = NUM_WORKERS * b_per_w
    mesh = plsc.VectorSubcoreMesh(core_axis_name="c", subcore_axis_name="s")

    @functools.partial(
        pl.kernel,
        mesh=mesh,
        compiler_params=pltpu.CompilerParams(use_tc_tiling_on_sc=False),
        out_type=jax.ShapeDtypeStruct((total, EMBED_DIM), jnp.float32),
        scratch_types=[
            pltpu.VMEM((steps, GATHER_ROWS), jnp.int32),
            pltpu.VMEM((NBUF, GATHER_ROWS, EMBED_DIM), jnp.float32),
        ] + [pltpu.SemaphoreType.DMA] * NBUF,
    )
    def k(idx_hbm, table_hbm, out_hbm, idx_v, rows_v, *sems):
        wid = lax.axis_index("s") * NUM_CORES + lax.axis_index("c")
        base = wid * b_per_w
        pltpu.sync_copy(idx_hbm.at[wid], idx_v)

        def start(g, b):
            pltpu.async_copy(table_hbm.at[idx_v.at[g]], rows_v.at[b], sems[b])

        def wait(b):
            pltpu.make_async_copy(
                table_hbm.at[pl.ds(0, GATHER_ROWS)], rows_v.at[b], sems[b]
            ).wait()

        def drain(g, b):
            wait(b)
            pltpu.sync_copy(
                rows_v.at[b],
                out_hbm.at[pl.ds(base + g * GATHER_ROWS, GATHER_ROWS)],
            )

        for b in range(NBUF):
            start(b, b)

        def outer(i, carry):
            g0 = i * NBUF
            for b in range(NBUF):
                drain(g0 + b, b)
                start(g0 + b + NBUF, b)
            return carry

        lax.fori_loop(0, steps // NBUF - 1, outer, 0)
        for b in range(NBUF):
            drain(steps - NBUF + b, b)

    return k(idx3, table)


def kernel(x, table):
    batch, hist = x.shape
    total = batch * hist
    b_per_w = total // NUM_WORKERS
    steps = b_per_w // GATHER_ROWS
    idx3 = x.reshape(NUM_WORKERS, steps, GATHER_ROWS).astype(jnp.int32)
    out = _sc_gather(idx3, table, b_per_w=b_per_w, steps=steps)
    return out.reshape(batch, hist, EMBED_DIM)
